# speedup vs baseline: 1.5244x; 1.5244x over previous
import functools

import jax
import jax.numpy as jnp
from jax import lax
from jax.experimental import pallas as pl
from jax.experimental.pallas import tpu as pltpu
from jax.experimental.pallas import tpu_sc as plsc

N, E, B, D = 10000, 320000, 16, 64
EPS = 1e-5
BLK = 2000
GRID = E // BLK


def _f32(*shape):
    return jax.ShapeDtypeStruct(shape, jnp.float32)


def _proj_body(x_ref, u_ref, batch_ref, W0_ref, b0_ref, prer_ref, prec_ref):
    x = x_ref[...]
    W0 = W0_ref[...]
    u = u_ref[...]
    ub = jnp.dot(u, W0[192:256], preferred_element_type=jnp.float32)
    onehot = (batch_ref[...] == lax.broadcasted_iota(jnp.int32, (N, B), 1)
              ).astype(jnp.float32)
    ubf = jnp.dot(onehot, ub, preferred_element_type=jnp.float32)
    prer_ref[...] = (jnp.dot(x, W0[0:64], preferred_element_type=jnp.float32)
                     + ubf + b0_ref[...])
    prec_ref[...] = jnp.dot(x, W0[64:128], preferred_element_type=jnp.float32)


def _proj(x, u, batch2d, W0, b0):
    return pl.pallas_call(
        _proj_body,
        out_shape=[_f32(N, D), _f32(N, D)],
    )(x, u, batch2d, W0, b0)


def _edge1_body(g_ref, ea_ref, Wc_ref, h_ref, st_ref, acc_ref):
    i = pl.program_id(0)

    @pl.when(i == 0)
    def _():
        acc_ref[...] = jnp.zeros_like(acc_ref)

    h = g_ref[...] + jnp.dot(ea_ref[...], Wc_ref[...],
                             preferred_element_type=jnp.float32)
    h = jnp.maximum(h, 0.0)
    h_ref[...] = h
    acc_ref[0:1, :] += jnp.sum(h, axis=0, keepdims=True)
    acc_ref[1:2, :] += jnp.sum(h * h, axis=0, keepdims=True)

    @pl.when(i == pl.num_programs(0) - 1)
    def _():
        st_ref[...] = acc_ref[0:2, :]


def _edge1(g, ea, Wc):
    return pl.pallas_call(
        _edge1_body,
        grid=(GRID,),
        in_specs=[pl.BlockSpec((BLK, D), lambda i: (i, 0)),
                  pl.BlockSpec((BLK, D), lambda i: (i, 0)),
                  pl.BlockSpec((D, D), lambda i: (0, 0))],
        out_specs=[pl.BlockSpec((BLK, D), lambda i: (i, 0)),
                   pl.BlockSpec((2, D), lambda i: (0, 0))],
        out_shape=[_f32(E, D), _f32(2, D)],
        scratch_shapes=[pltpu.VMEM((8, D), jnp.float32)],
    )(g, ea, Wc)


def _edge23_body(h_in_ref, st_in_ref, W_ref, b_ref, gam_ref, bet_ref,
                 gam3_ref, bet3_ref, h_ref, st_ref, aff_ref, acc_ref, *,
                 last):
    i = pl.program_id(0)

    @pl.when(i == 0)
    def _():
        acc_ref[...] = jnp.zeros_like(acc_ref)

    m = st_in_ref[0:1, :] * (1.0 / E)
    v = st_in_ref[1:2, :] * (1.0 / E) - m * m
    sc = gam_ref[...] * lax.rsqrt(v + EPS)
    hn = (h_in_ref[...] - m) * sc + bet_ref[...]
    h = jnp.dot(hn, W_ref[...], preferred_element_type=jnp.float32) + b_ref[...]
    h = jnp.maximum(h, 0.0)
    h_ref[...] = h
    acc_ref[0:1, :] += jnp.sum(h, axis=0, keepdims=True)
    acc_ref[1:2, :] += jnp.sum(h * h, axis=0, keepdims=True)

    @pl.when(i == pl.num_programs(0) - 1)
    def _():
        st_ref[...] = acc_ref[0:2, :]
        if last:
            m3 = acc_ref[0:1, :] * (1.0 / E)
            v3 = acc_ref[1:2, :] * (1.0 / E) - m3 * m3
            s3 = gam3_ref[...] * lax.rsqrt(v3 + EPS)
            aff_ref[0:1, :] = s3
            aff_ref[1:2, :] = bet3_ref[...] - m3 * s3
        else:
            aff_ref[...] = jnp.zeros_like(aff_ref)


def _edge23(h_in, st_in, W, b, gam, bet, gam3, bet3, last):
    return pl.pallas_call(
        functools.partial(_edge23_body, last=last),
        grid=(GRID,),
        in_specs=[pl.BlockSpec((BLK, D), lambda i: (i, 0)),
                  pl.BlockSpec((2, D), lambda i: (0, 0)),
                  pl.BlockSpec((D, D), lambda i: (0, 0)),
                  pl.BlockSpec((1, D), lambda i: (0, 0)),
                  pl.BlockSpec((1, D), lambda i: (0, 0)),
                  pl.BlockSpec((1, D), lambda i: (0, 0)),
                  pl.BlockSpec((1, D), lambda i: (0, 0)),
                  pl.BlockSpec((1, D), lambda i: (0, 0))],
        out_specs=[pl.BlockSpec((BLK, D), lambda i: (i, 0)),
                   pl.BlockSpec((2, D), lambda i: (0, 0)),
                   pl.BlockSpec((2, D), lambda i: (0, 0))],
        out_shape=[_f32(E, D), _f32(2, D), _f32(2, D)],
        scratch_shapes=[pltpu.VMEM((8, D), jnp.float32)],
    )(h_in, st_in, W, b, gam, bet, gam3, bet3)


def _node_global_body(x_ref, nsum_ref, cnt_ref, aff_ref, u_ref, batch_ref,
                      nW0_ref, nb0_ref, nWs_ref, nbs_ref, ng_ref, nbe_ref,
                      gW0_ref, gb0_ref, gWs_ref, gbs_ref, gg_ref, gbe_ref,
                      xout_ref, uout_ref):
    cnt = cnt_ref[0, :, 0:1] + cnt_ref[1, :, 0:1]
    nsum = nsum_ref[0] + nsum_ref[1]
    s3 = aff_ref[0:1, :]
    sh3 = aff_ref[1:2, :]
    ve = (nsum * s3 + cnt * sh3) / jnp.maximum(cnt, 1.0)

    onehot = (batch_ref[...] == lax.broadcasted_iota(jnp.int32, (N, B), 1)
              ).astype(jnp.float32)
    u = u_ref[...]
    ubf = jnp.dot(onehot, u, preferred_element_type=jnp.float32)

    x = x_ref[...]
    nW0 = nW0_ref[...]
    h = (jnp.dot(x, nW0[0:64], preferred_element_type=jnp.float32)
         + jnp.dot(ve, nW0[64:128], preferred_element_type=jnp.float32)
         + jnp.dot(ubf, nW0[128:192], preferred_element_type=jnp.float32)
         + nb0_ref[...])
    h = jnp.maximum(h, 0.0)
    for l in range(3):
        m = jnp.mean(h, axis=0, keepdims=True)
        v = jnp.mean(h * h, axis=0, keepdims=True) - m * m
        h = (h - m) * (ng_ref[l:l + 1, :] * lax.rsqrt(v + EPS)) \
            + nbe_ref[l:l + 1, :]
        if l < 2:
            h = jnp.dot(h, nWs_ref[l], preferred_element_type=jnp.float32) \
                + nbs_ref[l:l + 1, :]
            h = jnp.maximum(h, 0.0)
    xout_ref[...] = h

    dn = (((0,), (0,)), ((), ()))
    gcnt = lax.dot_general(onehot, jnp.ones((N, 1), jnp.float32), dn,
                           preferred_element_type=jnp.float32)
    gcnt = jnp.maximum(gcnt, 1.0)
    u_e = lax.dot_general(onehot, ve, dn,
                          preferred_element_type=jnp.float32) / gcnt
    u_v = lax.dot_general(onehot, h, dn,
                          preferred_element_type=jnp.float32) / gcnt
    gW0 = gW0_ref[...]
    hu = (jnp.dot(u_e, gW0[0:64], preferred_element_type=jnp.float32)
          + jnp.dot(u_v, gW0[64:128], preferred_element_type=jnp.float32)
          + jnp.dot(u, gW0[128:192], preferred_element_type=jnp.float32)
          + gb0_ref[...])
    hu = jnp.maximum(hu, 0.0)
    for l in range(3):
        m = jnp.mean(hu, axis=0, keepdims=True)
        v = jnp.mean(hu * hu, axis=0, keepdims=True) - m * m
        hu = (hu - m) * (gg_ref[l:l + 1, :] * lax.rsqrt(v + EPS)) \
            + gbe_ref[l:l + 1, :]
        if l < 2:
            hu = jnp.dot(hu, gWs_ref[l], preferred_element_type=jnp.float32) \
                + gbs_ref[l:l + 1, :]
            hu = jnp.maximum(hu, 0.0)
    uout_ref[...] = hu


def _node_global(x, nsum2, cnt2, aff, u, batch2d,
                 nW0, nb0, nWs, nbs, ng, nbe,
                 gW0, gb0, gWs, gbs, gg, gbe):
    return pl.pallas_call(
        _node_global_body,
        out_shape=[_f32(N, D), _f32(B, D)],
        compiler_params=pltpu.CompilerParams(
            vmem_limit_bytes=100 * 1024 * 1024),
    )(x, nsum2, cnt2, aff, u, batch2d,
      nW0, nb0, nWs, nbs, ng, nbe, gW0, gb0, gWs, gbs, gg, gbe)


def kernel(x, edge_attr, u, edge_index, batch,
           e_W0, e_b0, e_Ws, e_bs, e_g, e_be,
           n_W0, n_b0, n_Ws, n_bs, n_g, n_be,
           g_W0, g_b0, g_Ws, g_bs, g_g, g_be):
    row = edge_index[0]
    col = edge_index[1]
    batch2d = batch.reshape(N, 1)

    pre_row, pre_col = _proj(x, u, batch2d, e_W0, e_b0.reshape(1, D))

    g = pre_row[row] + pre_col[col]
    cnt2 = jax.ops.segment_sum(jnp.ones((E, 16), jnp.float32), row,
                               num_segments=N).reshape(1, N, 16)
    cnt2 = jnp.concatenate([cnt2, jnp.zeros_like(cnt2)], axis=0)

    h1, st1 = _edge1(g, edge_attr, e_W0[128:192])
    h2, st2, _ = _edge23(h1, st1, e_Ws[0], e_bs[0].reshape(1, D),
                         e_g[0].reshape(1, D), e_be[0].reshape(1, D),
                         e_g[2].reshape(1, D), e_be[2].reshape(1, D), False)
    h3, st3, aff = _edge23(h2, st2, e_Ws[1], e_bs[1].reshape(1, D),
                           e_g[1].reshape(1, D), e_be[1].reshape(1, D),
                           e_g[2].reshape(1, D), e_be[2].reshape(1, D), True)

    e_out = h3 * aff[0:1] + aff[1:2]
    nsum2 = jax.ops.segment_sum(h3, row, num_segments=N).reshape(1, N, D)
    nsum2 = jnp.concatenate([nsum2, jnp.zeros_like(nsum2)], axis=0)

    x_out, u_out = _node_global(
        x, nsum2, cnt2, aff, u, batch2d,
        n_W0, n_b0.reshape(1, D), n_Ws, n_bs, n_g, n_be,
        g_W0, g_b0.reshape(1, D), g_Ws, g_bs, g_g, g_be)
    return (x_out, e_out, u_out)
